# baseline (speedup 1.0000x reference)
import jax
import jax.numpy as jnp
from jax import lax
from jax.experimental import pallas as pl
from jax.experimental.pallas import tpu as pltpu
from jax.experimental.pallas import tpu_sc as plsc

_SEQ = 8192
_D = 768
_B = 4
_B_SC = 2

_NC = 2
_NS = 16
_NW = _NC * _NS
_ROWS_W = _SEQ // _NW
_R = 16
_NJ = _ROWS_W // _R
_NITEMS = _NJ * _B_SC
_LANES = 16

_BS_TC = 256


def _sc_body(in_hbm, tab_hbm, out_hbm, tab_v, in_v, out_v,
             tab_sem, in_sem, out_sem):
    wid = lax.axis_index("s") * _NC + lax.axis_index("c")
    rbase = wid * _ROWS_W

    def tab_copy(j, jp):
        return pltpu.make_async_copy(
            tab_hbm.at[pl.ds(rbase + j * _R, _R)], tab_v.at[jp], tab_sem.at[jp])

    def in_copy(t, p):
        r0 = rbase + (t // _B_SC) * _R
        return pltpu.make_async_copy(
            in_hbm.at[t % _B_SC, pl.ds(r0, _R)], in_v.at[p], in_sem.at[p])

    def out_copy(t, p):
        r0 = rbase + (t // _B_SC) * _R
        return pltpu.make_async_copy(
            out_v.at[p], out_hbm.at[t % _B_SC, pl.ds(r0, _R)], out_sem.at[p])

    tab_copy(0, 0).start()
    in_copy(0, 0).start()

    def item(t, _):
        j = t // _B_SC
        b = t % _B_SC
        p = t % 2
        jp = j % 2

        @pl.when(t + 1 < _NITEMS)
        def _():
            in_copy(t + 1, (t + 1) % 2).start()

        @pl.when((b == 0) & (j + 1 < _NJ))
        def _():
            tab_copy(j + 1, (j + 1) % 2).start()

        in_copy(t, p).wait()

        @pl.when(b == 0)
        def _():
            tab_copy(j, jp).wait()

        @pl.when(t >= 2)
        def _():
            out_copy(t - 2, p).wait()

        @plsc.parallel_loop(0, _R, step=1)
        def _(r):
            for c in range(0, _D, _LANES):
                sl = pl.ds(c, _LANES)
                out_v[p, r, sl] = in_v[p, r, sl] + tab_v[jp, r, sl]

        out_copy(t, p).start()
        return 0

    lax.fori_loop(0, _NITEMS, item, 0)

    out_copy(_NITEMS - 2, (_NITEMS - 2) % 2).wait()
    out_copy(_NITEMS - 1, (_NITEMS - 1) % 2).wait()


def _tc_body(in_ref, tab_ref, out_ref):
    out_ref[...] = in_ref[...] + tab_ref[...][None, :, :]


@jax.jit
def kernel(inputs, pos_table):
    mesh = plsc.VectorSubcoreMesh(core_axis_name="c", subcore_axis_name="s")
    sc_k = pl.kernel(
        _sc_body,
        out_type=jax.ShapeDtypeStruct((_B_SC, _SEQ, _D), jnp.float32),
        mesh=mesh,
        scratch_types=[
            pltpu.VMEM((2, _R, _D), jnp.float32),
            pltpu.VMEM((2, _R, _D), jnp.float32),
            pltpu.VMEM((2, _R, _D), jnp.float32),
            pltpu.SemaphoreType.DMA((2,)),
            pltpu.SemaphoreType.DMA((2,)),
            pltpu.SemaphoreType.DMA((2,)),
        ],
    )
    sc_out = sc_k(inputs, pos_table)

    n_tc = _B - _B_SC
    tc_out = pl.pallas_call(
        _tc_body,
        grid=(_SEQ // _BS_TC,),
        in_specs=[
            pl.BlockSpec((n_tc, _BS_TC, _D), lambda i: (_B_SC // n_tc, i, 0)),
            pl.BlockSpec((_BS_TC, _D), lambda i: (i, 0)),
        ],
        out_specs=pl.BlockSpec((n_tc, _BS_TC, _D), lambda i: (0, i, 0)),
        out_shape=jax.ShapeDtypeStruct((n_tc, _SEQ, _D), jnp.float32),
    )(inputs, pos_table)

    return jnp.concatenate([sc_out, tc_out], axis=0)


# speedup vs baseline: 1.2110x; 1.2110x over previous
import jax
import jax.numpy as jnp
from jax import lax
from jax.experimental import pallas as pl
from jax.experimental.pallas import tpu as pltpu
from jax.experimental.pallas import tpu_sc as plsc

_SEQ = 8192
_D = 768
_B = 4
_B_SC = 3

_NC = 2
_NS = 16
_NW = _NC * _NS
_ROWS_W = _SEQ // _NW
_R = 16
_NJ = _ROWS_W // _R
_NITEMS = _NJ * _B_SC
_LANES = 16

_BS_TC = 256


def _sc_body(in_hbm, tab_hbm, out_hbm, tab_v, in_v, out_v,
             tab_sem, in_sem, out_sem):
    wid = lax.axis_index("s") * _NC + lax.axis_index("c")
    rbase = wid * _ROWS_W

    def tab_copy(j, jp):
        return pltpu.make_async_copy(
            tab_hbm.at[pl.ds(rbase + j * _R, _R)], tab_v.at[jp], tab_sem.at[jp])

    def in_copy(t, p):
        r0 = rbase + (t // _B_SC) * _R
        return pltpu.make_async_copy(
            in_hbm.at[t % _B_SC, pl.ds(r0, _R)], in_v.at[p], in_sem.at[p])

    def out_copy(t, p):
        r0 = rbase + (t // _B_SC) * _R
        return pltpu.make_async_copy(
            out_v.at[p], out_hbm.at[t % _B_SC, pl.ds(r0, _R)], out_sem.at[p])

    tab_copy(0, 0).start()
    in_copy(0, 0).start()

    def item(t, _):
        j = t // _B_SC
        b = t % _B_SC
        p = t % 2
        jp = j % 2

        @pl.when(t + 1 < _NITEMS)
        def _():
            in_copy(t + 1, (t + 1) % 2).start()

        @pl.when((b == 0) & (j + 1 < _NJ))
        def _():
            tab_copy(j + 1, (j + 1) % 2).start()

        in_copy(t, p).wait()

        @pl.when(b == 0)
        def _():
            tab_copy(j, jp).wait()

        @pl.when(t >= 2)
        def _():
            out_copy(t - 2, p).wait()

        @plsc.parallel_loop(0, _R, step=1)
        def _(r):
            for c in range(0, _D, _LANES):
                sl = pl.ds(c, _LANES)
                out_v[p, r, sl] = in_v[p, r, sl] + tab_v[jp, r, sl]

        out_copy(t, p).start()
        return 0

    lax.fori_loop(0, _NITEMS, item, 0)

    out_copy(_NITEMS - 2, (_NITEMS - 2) % 2).wait()
    out_copy(_NITEMS - 1, (_NITEMS - 1) % 2).wait()


def _tc_body(in_ref, tab_ref, out_ref):
    out_ref[...] = in_ref[...] + tab_ref[...][None, :, :]


def _merge_body(full_ref, tc_ref, out_ref):
    out_ref[...] = tc_ref[...]


@jax.jit
def kernel(inputs, pos_table):
    mesh = plsc.VectorSubcoreMesh(core_axis_name="c", subcore_axis_name="s")
    sc_k = pl.kernel(
        _sc_body,
        out_type=jax.ShapeDtypeStruct((_B, _SEQ, _D), jnp.float32),
        mesh=mesh,
        scratch_types=[
            pltpu.VMEM((2, _R, _D), jnp.float32),
            pltpu.VMEM((2, _R, _D), jnp.float32),
            pltpu.VMEM((2, _R, _D), jnp.float32),
            pltpu.SemaphoreType.DMA((2,)),
            pltpu.SemaphoreType.DMA((2,)),
            pltpu.SemaphoreType.DMA((2,)),
        ],
    )
    sc_out = sc_k(inputs, pos_table)

    n_tc = _B - _B_SC
    tc_out = pl.pallas_call(
        _tc_body,
        grid=(_SEQ // _BS_TC,),
        in_specs=[
            pl.BlockSpec((n_tc, _BS_TC, _D), lambda i: (_B_SC // n_tc, i, 0)),
            pl.BlockSpec((_BS_TC, _D), lambda i: (i, 0)),
        ],
        out_specs=pl.BlockSpec((n_tc, _BS_TC, _D), lambda i: (0, i, 0)),
        out_shape=jax.ShapeDtypeStruct((n_tc, _SEQ, _D), jnp.float32),
    )(inputs, pos_table)

    return pl.pallas_call(
        _merge_body,
        grid=(_SEQ // _BS_TC,),
        in_specs=[
            pl.BlockSpec(memory_space=pl.ANY),
            pl.BlockSpec((n_tc, _BS_TC, _D), lambda i: (0, i, 0)),
        ],
        out_specs=pl.BlockSpec((n_tc, _BS_TC, _D), lambda i: (_B_SC // n_tc, i, 0)),
        out_shape=jax.ShapeDtypeStruct((_B, _SEQ, _D), jnp.float32),
        input_output_aliases={0: 0},
    )(sc_out, tc_out)


# speedup vs baseline: 1.4690x; 1.2130x over previous
import jax
import jax.numpy as jnp
from jax import lax
from jax.experimental import pallas as pl
from jax.experimental.pallas import tpu as pltpu
from jax.experimental.pallas import tpu_sc as plsc

_SEQ = 8192
_D = 768
_B = 4

_NC = 2
_NS = 16
_NW = _NC * _NS
_ROWS_W = _SEQ // _NW
_R = 16
_NJ = _ROWS_W // _R
_NITEMS = _NJ * _B
_LANES = 16


def _sc_body(in_hbm, tab_hbm, out_hbm, tab_v, in_v, out_v,
             tab_sem, in_sem, out_sem):
    wid = lax.axis_index("s") * _NC + lax.axis_index("c")
    rbase = wid * _ROWS_W

    def tab_copy(j, jp):
        return pltpu.make_async_copy(
            tab_hbm.at[pl.ds(rbase + j * _R, _R)], tab_v.at[jp], tab_sem.at[jp])

    def in_copy(t, p):
        r0 = rbase + (t // _B) * _R
        return pltpu.make_async_copy(
            in_hbm.at[t % _B, pl.ds(r0, _R)], in_v.at[p], in_sem.at[p])

    def out_copy(t, p):
        r0 = rbase + (t // _B) * _R
        return pltpu.make_async_copy(
            out_v.at[p], out_hbm.at[t % _B, pl.ds(r0, _R)], out_sem.at[p])

    tab_copy(0, 0).start()
    in_copy(0, 0).start()

    def item(t, _):
        j = t // _B
        b = t % _B
        p = t % 2
        jp = j % 2

        @pl.when(t + 1 < _NITEMS)
        def _():
            in_copy(t + 1, (t + 1) % 2).start()

        @pl.when((b == 0) & (j + 1 < _NJ))
        def _():
            tab_copy(j + 1, (j + 1) % 2).start()

        in_copy(t, p).wait()

        @pl.when(b == 0)
        def _():
            tab_copy(j, jp).wait()

        @pl.when(t >= 2)
        def _():
            out_copy(t - 2, p).wait()

        @plsc.parallel_loop(0, _R, step=1)
        def _(r):
            for c in range(0, _D, _LANES):
                sl = pl.ds(c, _LANES)
                out_v[p, r, sl] = in_v[p, r, sl] + tab_v[jp, r, sl]

        out_copy(t, p).start()
        return 0

    lax.fori_loop(0, _NITEMS, item, 0)

    out_copy(_NITEMS - 2, 0).wait()
    out_copy(_NITEMS - 1, 1).wait()


@jax.jit
def kernel(inputs, pos_table):
    mesh = plsc.VectorSubcoreMesh(core_axis_name="c", subcore_axis_name="s")
    k = pl.kernel(
        _sc_body,
        out_type=jax.ShapeDtypeStruct((_B, _SEQ, _D), jnp.float32),
        mesh=mesh,
        scratch_types=[
            pltpu.VMEM((2, _R, _D), jnp.float32),
            pltpu.VMEM((2, _R, _D), jnp.float32),
            pltpu.VMEM((2, _R, _D), jnp.float32),
            pltpu.SemaphoreType.DMA((2,)),
            pltpu.SemaphoreType.DMA((2,)),
            pltpu.SemaphoreType.DMA((2,)),
        ],
        compiler_params=pltpu.CompilerParams(
            disable_bounds_checks=True,
            disable_semaphore_checks=True,
        ),
    )
    return k(inputs, pos_table)
